# baseline (device time: 47434 ns/iter reference)
import jax
import jax.numpy as jnp
from jax import lax
from jax.experimental import pallas as pl
from jax.experimental.pallas import tpu as pltpu

B = 8
NB = 512
BS = 16
H = 8
D = 128
NKEY = NB * BS
HALF = NKEY // 2
LH = H // 2
NPEER = 3


def kernel(Q, K, V, bt, lens):
    Qt = jnp.transpose(Q.reshape(B, H, D), (1, 0, 2))
    lens2 = lens.reshape(B, 1)
    K2 = K.reshape(NKEY, H, D)
    V2 = V.reshape(NKEY, H, D)

    def body(q_ref, k_ref, v_ref, bt_ref, lens_ref, out_ref,
             wide, kh_buf, vh_buf, msg_send, msg_recv,
             w_sems, d_sems, send_sems, recv_sems):
        my_x = lax.axis_index("x")
        my_y = lax.axis_index("y")
        peers = (
            (my_x, 1 - my_y),
            (1 - my_x, my_y),
            (1 - my_x, 1 - my_y),
        )
        hbase = LH * my_x

        barrier_sem = pltpu.get_barrier_semaphore()
        for p in peers:
            pl.semaphore_signal(
                barrier_sem, inc=1,
                device_id=p, device_id_type=pl.DeviceIdType.MESH,
            )

        def wide_dma(src_ref, half):
            rows = pl.ds(half * HALF, HALF)
            return pltpu.make_async_copy(
                src_ref.at[rows, pl.ds(hbase, LH), :],
                wide.at[rows],
                w_sems.at[half],
            )

        def deint_dma(dst, i, half, which):
            rows = pl.ds(half * HALF, HALF)
            return pltpu.make_async_copy(
                wide.at[rows, i, :],
                dst.at[i, rows],
                d_sems.at[which, i, half],
            )

        def peer_rdma(p, i):
            return pltpu.make_async_remote_copy(
                src_ref=msg_send.at[i], dst_ref=msg_recv.at[p, i],
                send_sem=send_sems.at[p, i], recv_sem=recv_sems.at[p, i],
                device_id=peers[p], device_id_type=pl.DeviceIdType.MESH,
            )

        wide_dma(k_ref, 0).start()
        wide_dma(k_ref, 1).start()

        bt_v = bt_ref[...]
        lens_v = lens_ref[...]
        jcol = lax.broadcasted_iota(jnp.int32, (B, NB), 1)
        bt_m = jnp.where(jcol < lens_v, bt_v, -1)
        page_id = (
            lax.broadcasted_iota(jnp.int32, (B, NB, NB), 2) + my_y * NB
        )
        hit = bt_m[:, :, None] == page_id
        counts = jnp.sum(hit.astype(jnp.float32), axis=1)
        counts_keys = jnp.broadcast_to(
            counts[:, :, None], (B, NB, BS)
        ).reshape(B, NKEY)

        for half in range(2):
            wide_dma(k_ref, half).wait()
            for i in range(LH):
                deint_dma(kh_buf, i, half, 0).start()
        for i in range(LH):
            for half in range(2):
                deint_dma(kh_buf, i, half, 0).wait()

        wide_dma(v_ref, 0).start()
        wide_dma(v_ref, 1).start()

        scale = D ** -0.5
        e_acc = [None] * LH
        l_acc = [None] * LH
        for i in range(LH):
            qh = q_ref[pl.ds(hbase + i, 1)].reshape(B, D) * scale
            s = lax.dot_general(
                qh, kh_buf[i], (((1,), (1,)), ((), ())),
                preferred_element_type=jnp.float32,
            )
            e_acc[i] = jnp.exp(s) * counts_keys
            l_acc[i] = jnp.sum(e_acc[i], axis=1, keepdims=True)

        for half in range(2):
            wide_dma(v_ref, half).wait()
            for i in range(LH):
                deint_dma(vh_buf, i, half, 1).start()
        for i in range(LH):
            for half in range(2):
                deint_dma(vh_buf, i, half, 1).wait()

        for i in range(LH):
            o_h = lax.dot_general(
                e_acc[i], vh_buf[i], (((1,), (0,)), ((), ())),
                preferred_element_type=jnp.float32,
            )
            msg_send[i, 0:B, :] = o_h
            msg_send[i, B:2 * B, :] = jnp.broadcast_to(l_acc[i], (B, D))
            if i == 0:
                pl.semaphore_wait(barrier_sem, NPEER)
            for p in range(NPEER):
                peer_rdma(p, i).start()

        other_base = LH * (1 - my_x)
        for i in range(LH):
            peer_rdma(0, i).wait()
            o_r = msg_recv[0, i, 0:B, :]
            l_r = msg_recv[0, i, B:2 * B, 0:1]
            o_f = (msg_send[i, 0:B, :] + o_r) / (l_acc[i] + l_r)
            out_ref[pl.ds(hbase + i, 1)] = o_f[None, :, :]

        for i in range(LH):
            peer_rdma(1, i).wait()
            peer_rdma(2, i).wait()
            o_a = msg_recv[1, i, 0:B, :]
            l_a = msg_recv[1, i, B:2 * B, 0:1]
            o_b = msg_recv[2, i, 0:B, :]
            l_b = msg_recv[2, i, B:2 * B, 0:1]
            o_f = (o_a + o_b) / (l_a + l_b)
            out_ref[pl.ds(other_base + i, 1)] = o_f[None, :, :]

    out = pl.pallas_call(
        body,
        out_shape=jax.ShapeDtypeStruct((H, B, D), jnp.float32),
        in_specs=[
            pl.BlockSpec(memory_space=pltpu.VMEM),
            pl.BlockSpec(memory_space=pl.ANY),
            pl.BlockSpec(memory_space=pl.ANY),
            pl.BlockSpec(memory_space=pltpu.VMEM),
            pl.BlockSpec(memory_space=pltpu.VMEM),
        ],
        out_specs=pl.BlockSpec(memory_space=pltpu.VMEM),
        scratch_shapes=[
            pltpu.VMEM((NKEY, LH, D), jnp.float32),
            pltpu.VMEM((LH, NKEY, D), jnp.float32),
            pltpu.VMEM((LH, NKEY, D), jnp.float32),
            pltpu.VMEM((LH, 2 * B, D), jnp.float32),
            pltpu.VMEM((NPEER, LH, 2 * B, D), jnp.float32),
            pltpu.SemaphoreType.DMA((2,)),
            pltpu.SemaphoreType.DMA((2, LH, 2)),
            pltpu.SemaphoreType.DMA((NPEER, LH)),
            pltpu.SemaphoreType.DMA((NPEER, LH)),
        ],
        compiler_params=pltpu.CompilerParams(
            collective_id=0,
            vmem_limit_bytes=62 * 1024 * 1024,
        ),
    )(Qt, K2, V2, bt, lens2)

    return jnp.transpose(out, (1, 0, 2)).reshape(B, 1, H, D)


# device time: 24483 ns/iter; 1.9374x vs baseline; 1.9374x over previous
import jax
import jax.numpy as jnp
from jax import lax
from jax.experimental import pallas as pl
from jax.experimental.pallas import tpu as pltpu

B = 8
NB = 512
BS = 16
H = 8
D = 128
NKEY = NB * BS
LH = H // 2
NPEER = 3


def kernel(Q, K, V, bt, lens):
    Qt = jnp.transpose(Q.reshape(B, H, D), (1, 0, 2))
    lens2 = lens.reshape(B, 1)
    K2 = K.reshape(NKEY, H, D)
    V2 = V.reshape(NKEY, H, D)

    def body(q_ref, k_ref, v_ref, bt_ref, lens_ref, out_ref,
             k_buf, v_buf, msg_send, msg_recv,
             k_sems, v_sems, send_sems, recv_sems):
        my_x = lax.axis_index("x")
        my_y = lax.axis_index("y")
        peers = (
            (my_x, 1 - my_y),
            (1 - my_x, my_y),
            (1 - my_x, 1 - my_y),
        )
        hbase = LH * my_x

        barrier_sem = pltpu.get_barrier_semaphore()
        for p in peers:
            pl.semaphore_signal(
                barrier_sem, inc=1,
                device_id=p, device_id_type=pl.DeviceIdType.MESH,
            )

        def kv_dma(i):
            h = hbase + i
            return (
                pltpu.make_async_copy(
                    k_ref.at[:, h, :], k_buf.at[i], k_sems.at[i]
                ),
                pltpu.make_async_copy(
                    v_ref.at[:, h, :], v_buf.at[i], v_sems.at[i]
                ),
            )

        def peer_rdma(p, i):
            return pltpu.make_async_remote_copy(
                src_ref=msg_send.at[i], dst_ref=msg_recv.at[p, i],
                send_sem=send_sems.at[p, i], recv_sem=recv_sems.at[p, i],
                device_id=peers[p], device_id_type=pl.DeviceIdType.MESH,
            )

        for i0 in range(LH):
            kd0, vd0 = kv_dma(i0)
            kd0.start()
            vd0.start()

        bt_v = bt_ref[...]
        lens_v = lens_ref[...]
        jcol = lax.broadcasted_iota(jnp.int32, (B, NB), 1)
        bt_m = jnp.where(jcol < lens_v, bt_v, -1)
        page_id = (
            lax.broadcasted_iota(jnp.int32, (B, NB, NB), 2) + my_y * NB
        )
        hit = bt_m[:, :, None] == page_id
        counts = jnp.sum(hit.astype(jnp.float32), axis=1)
        counts_keys = jnp.broadcast_to(
            counts[:, :, None], (B, NB, BS)
        ).reshape(B, NKEY)

        scale = D ** -0.5

        l_acc = [None] * LH
        for i in range(LH):
            kw, vw = kv_dma(i)
            kw.wait()
            vw.wait()

            kh = k_buf[i]
            vh = v_buf[i]
            qh = q_ref[pl.ds(hbase + i, 1)].reshape(B, D) * scale
            s = lax.dot_general(
                qh, kh, (((1,), (1,)), ((), ())),
                preferred_element_type=jnp.float32,
            )
            e_h = jnp.exp(s) * counts_keys
            l_h = jnp.sum(e_h, axis=1, keepdims=True)
            o_h = lax.dot_general(
                e_h, vh, (((1,), (0,)), ((), ())),
                preferred_element_type=jnp.float32,
            )
            l_acc[i] = l_h

            msg_send[i, 0:B, :] = o_h
            msg_send[i, B:2 * B, :] = jnp.broadcast_to(l_h, (B, D))
            if i == 0:
                pl.semaphore_wait(barrier_sem, NPEER)
            for p in range(NPEER):
                peer_rdma(p, i).start()

        other_base = LH * (1 - my_x)
        for i in range(LH):
            peer_rdma(0, i).wait()
            o_r = msg_recv[0, i, 0:B, :]
            l_r = msg_recv[0, i, B:2 * B, 0:1]
            o_f = (msg_send[i, 0:B, :] + o_r) / (l_acc[i] + l_r)
            out_ref[pl.ds(hbase + i, 1)] = o_f[None, :, :]

        for i in range(LH):
            peer_rdma(1, i).wait()
            peer_rdma(2, i).wait()
            o_a = msg_recv[1, i, 0:B, :]
            l_a = msg_recv[1, i, B:2 * B, 0:1]
            o_b = msg_recv[2, i, 0:B, :]
            l_b = msg_recv[2, i, B:2 * B, 0:1]
            o_f = (o_a + o_b) / (l_a + l_b)
            out_ref[pl.ds(other_base + i, 1)] = o_f[None, :, :]

    out = pl.pallas_call(
        body,
        out_shape=jax.ShapeDtypeStruct((H, B, D), jnp.float32),
        in_specs=[
            pl.BlockSpec(memory_space=pltpu.VMEM),
            pl.BlockSpec(memory_space=pl.ANY),
            pl.BlockSpec(memory_space=pl.ANY),
            pl.BlockSpec(memory_space=pltpu.VMEM),
            pl.BlockSpec(memory_space=pltpu.VMEM),
        ],
        out_specs=pl.BlockSpec(memory_space=pltpu.VMEM),
        scratch_shapes=[
            pltpu.VMEM((LH, NKEY, D), jnp.float32),
            pltpu.VMEM((LH, NKEY, D), jnp.float32),
            pltpu.VMEM((LH, 2 * B, D), jnp.float32),
            pltpu.VMEM((NPEER, LH, 2 * B, D), jnp.float32),
            pltpu.SemaphoreType.DMA((LH,)),
            pltpu.SemaphoreType.DMA((LH,)),
            pltpu.SemaphoreType.DMA((NPEER, LH)),
            pltpu.SemaphoreType.DMA((NPEER, LH)),
        ],
        compiler_params=pltpu.CompilerParams(
            collective_id=0,
            vmem_limit_bytes=60 * 1024 * 1024,
        ),
    )(Qt, K2, V2, bt, lens2)

    return jnp.transpose(out, (1, 0, 2)).reshape(B, 1, H, D)
